# Initial kernel scaffold; baseline (speedup 1.0000x reference)
import jax
import jax.numpy as jnp
from jax.experimental import pallas as pl


def kernel(t0, t1, t2, t3, t4, img, nlp, nlp_revert_padding_mask, W_temporal_mod, W_total_mod, global_token):
    raise NotImplementedError("write your pallas kernel here")


# speedup vs baseline: 7.5506x; 7.5506x over previous
import functools

import jax
import jax.numpy as jnp
import numpy as np
from jax.experimental import pallas as pl
from jax.experimental.pallas import tpu as pltpu

D_MODEL = 768
B = 4
L_T = 2048


def _pe_table_np(d_model, max_len):
    position = np.arange(max_len)[:, None].astype(np.float32)
    div_term = np.exp(np.arange(0, d_model, 2).astype(np.float32) * (-np.log(10000.0) / d_model))
    pe = np.zeros((max_len, d_model), np.float32)
    pe[:, 0::2] = np.sin(position * div_term)
    pe[:, 1::2] = np.cos(position * div_term)
    return pe


def _1d_sincos_np(embed_dim, pos):
    omega = np.arange(embed_dim // 2, dtype=np.float32) / (embed_dim / 2.0)
    omega = 1.0 / 10000 ** omega
    pos = pos.reshape(-1)
    out = np.einsum('m,d->md', pos, omega)
    return np.concatenate([np.sin(out), np.cos(out)], axis=1)


def _2d_sincos_np(embed_dim, grid_size, cls_token=True):
    grid_h = np.arange(grid_size, dtype=np.float32)
    grid_w = np.arange(grid_size, dtype=np.float32)
    grid = np.meshgrid(grid_w, grid_h)
    grid = np.stack(grid, axis=0).reshape([2, 1, grid_size, grid_size])
    emb_h = _1d_sincos_np(embed_dim // 2, grid[0])
    emb_w = _1d_sincos_np(embed_dim // 2, grid[1])
    pos = np.concatenate([emb_h, emb_w], axis=1).astype(np.float32)
    if cls_token:
        pos = np.concatenate([np.zeros([1, embed_dim], np.float32), pos], axis=0)
    return pos


@functools.lru_cache(maxsize=None)
def _consts():
    mkey = jax.random.key(42)

    def idx(key, shape, num_remain):
        noise = jax.random.uniform(key, shape)
        shuffle = jnp.argsort(noise, axis=-1)
        remain = shuffle[..., :num_remain]
        masked = shuffle[..., num_remain:]
        revert = jnp.argsort(shuffle, axis=-1)
        return (np.asarray(jax.device_get(remain)),
                np.asarray(jax.device_get(masked)),
                np.asarray(jax.device_get(revert)))

    r_t, m_t, rv_t = idx(jax.random.fold_in(mkey, 0), (B, L_T, 4), 2)
    r_i, m_i, rv_i = idx(jax.random.fold_in(mkey, 1), (B, 196), 49)
    r_n, m_n, rv_n = idx(jax.random.fold_in(mkey, 2), (B, 512), 128)

    pe = _pe_table_np(D_MODEL, 4096)
    pos2d = _2d_sincos_np(D_MODEL, 14, True)

    S = np.zeros((B, L_T, 8), np.float32)
    for j in range(2):
        for mm in range(4):
            S[:, :, j * 4 + mm] = (r_t[:, :, j] == mm).astype(np.float32)

    Gi = np.zeros((B, 49, 196), np.float32)
    Gi[np.arange(B)[:, None], np.arange(49)[None, :], r_i] = 1.0
    Ci = pos2d[1:][r_i]

    Gn = np.zeros((B, 128, 512), np.float32)
    Gn[np.arange(B)[:, None], np.arange(128)[None, :], r_n] = 1.0
    Cn = pe[1:513][r_n]
    Mn = np.zeros((B, 384, 512), np.float32)
    Mn[np.arange(B)[:, None], np.arange(384)[None, :], m_n] = 1.0

    return dict(
        r_t=r_t, m_t=m_t, rv_t=rv_t,
        m_i=m_i, rv_i=rv_i, m_n=m_n, rv_n=rv_n,
        pe=pe, pos2d=pos2d,
        S=S, Gi=Gi, Ci=Ci, Gn=Gn, Cn=Cn, Mn=Mn,
    )


_C = _consts()


_LBLK = 512


def _temporal_body(t0, t1, t2, t3, t4, s, pe, w, o):
    peb = pe[...]
    wv = w[...]
    sb = s[0]
    out0 = t0[0] + peb + wv[0]
    x1 = (sb[:, 0:1] * t1[0] + sb[:, 1:2] * t2[0]
          + sb[:, 2:3] * t3[0] + sb[:, 3:4] * t4[0])
    w1 = jax.lax.dot_general(sb[:, 0:4], wv[1:5], (((1,), (0,)), ((), ())),
                             preferred_element_type=jnp.float32)
    x2 = (sb[:, 4:5] * t1[0] + sb[:, 5:6] * t2[0]
          + sb[:, 6:7] * t3[0] + sb[:, 7:8] * t4[0])
    w2 = jax.lax.dot_general(sb[:, 4:8], wv[1:5], (((1,), (0,)), ((), ())),
                             preferred_element_type=jnp.float32)
    o[0] = jnp.stack([out0, x1 + w1 + peb, x2 + w2 + peb], axis=1)


def _temporal(t0, t1, t2, t3, t4, S, pe_t, w_all):
    tb = pl.BlockSpec((1, _LBLK, D_MODEL), lambda b, l: (b, l, 0))
    return pl.pallas_call(
        _temporal_body,
        grid=(B, L_T // _LBLK),
        in_specs=[tb, tb, tb, tb, tb,
                  pl.BlockSpec((1, _LBLK, 8), lambda b, l: (b, l, 0)),
                  pl.BlockSpec((_LBLK, D_MODEL), lambda b, l: (l, 0)),
                  pl.BlockSpec((5, D_MODEL), lambda b, l: (0, 0))],
        out_specs=pl.BlockSpec((1, _LBLK, 3, D_MODEL), lambda b, l: (b, l, 0, 0)),
        out_shape=jax.ShapeDtypeStruct((B, L_T, 3, D_MODEL), jnp.float32),
    )(t0, t1, t2, t3, t4, S, pe_t, w_all)


def _img_body(img, gi, ci, gtp, o):
    rows = jax.lax.dot_general(gi[0], img[0], (((1,), (0,)), ((), ())),
                               preferred_element_type=jnp.float32) + ci[0]
    o[0] = jnp.concatenate([gtp[...], rows], axis=0)


def _img_sr(img, Gi, Ci, gtp):
    return pl.pallas_call(
        _img_body,
        grid=(B,),
        in_specs=[pl.BlockSpec((1, 196, D_MODEL), lambda b: (b, 0, 0)),
                  pl.BlockSpec((1, 49, 196), lambda b: (b, 0, 0)),
                  pl.BlockSpec((1, 49, D_MODEL), lambda b: (b, 0, 0)),
                  pl.BlockSpec((1, D_MODEL), lambda b: (0, 0))],
        out_specs=pl.BlockSpec((1, 50, D_MODEL), lambda b: (b, 0, 0)),
        out_shape=jax.ShapeDtypeStruct((B, 50, D_MODEL), jnp.float32),
    )(img, Gi, Ci, gtp)


def _nlp_body(nlp, gn, cn, mn, msk, gtp, o_sr, o_rpm, o_mpm):
    rows = jax.lax.dot_general(gn[0], nlp[0], (((1,), (0,)), ((), ())),
                               preferred_element_type=jnp.float32) + cn[0]
    o_sr[0] = jnp.concatenate([gtp[...], rows], axis=0)
    mrow = msk[0]
    mv = mrow[:, 1:]
    g0 = mrow[:, 0:1]
    rpm = jax.lax.dot_general(mv, gn[0], (((1,), (1,)), ((), ())),
                              preferred_element_type=jnp.float32)
    mpm = jax.lax.dot_general(mv, mn[0], (((1,), (1,)), ((), ())),
                              preferred_element_type=jnp.float32)
    o_rpm[0] = jnp.concatenate([g0, rpm], axis=1)
    o_mpm[0] = jnp.concatenate([g0, mpm], axis=1)


def _nlp_all(nlp, Gn, Cn, Mn, mask3, gtp):
    return pl.pallas_call(
        _nlp_body,
        grid=(B,),
        in_specs=[pl.BlockSpec((1, 512, D_MODEL), lambda b: (b, 0, 0)),
                  pl.BlockSpec((1, 128, 512), lambda b: (b, 0, 0)),
                  pl.BlockSpec((1, 128, D_MODEL), lambda b: (b, 0, 0)),
                  pl.BlockSpec((1, 384, 512), lambda b: (b, 0, 0)),
                  pl.BlockSpec((1, 1, 513), lambda b: (b, 0, 0)),
                  pl.BlockSpec((1, D_MODEL), lambda b: (0, 0))],
        out_specs=[pl.BlockSpec((1, 129, D_MODEL), lambda b: (b, 0, 0)),
                   pl.BlockSpec((1, 1, 129), lambda b: (b, 0, 0)),
                   pl.BlockSpec((1, 1, 385), lambda b: (b, 0, 0))],
        out_shape=[jax.ShapeDtypeStruct((B, 129, D_MODEL), jnp.float32),
                   jax.ShapeDtypeStruct((B, 1, 129), jnp.float32),
                   jax.ShapeDtypeStruct((B, 1, 385), jnp.float32)],
    )(nlp, Gn, Cn, Mn, mask3, gtp)


def kernel(t0, t1, t2, t3, t4, img, nlp, nlp_revert_padding_mask,
           W_temporal_mod, W_total_mod, global_token):
    c = _C
    pe_t = jnp.asarray(c['pe'][:L_T])
    w_all = W_temporal_mod + W_total_mod[0][None, :]

    temporal = _temporal(t0, t1, t2, t3, t4, jnp.asarray(c['S']), pe_t, w_all)

    gt = global_token[0]
    gtp_img = gt + jnp.asarray(c['pos2d'][0:1])
    img_sr = _img_sr(img, jnp.asarray(c['Gi']), jnp.asarray(c['Ci']), gtp_img)

    gtp_nlp = gt + jnp.asarray(c['pe'][0:1])
    mask3 = nlp_revert_padding_mask[:, None, :]
    nlp_sr, rpm3, mpm3 = _nlp_all(nlp, jnp.asarray(c['Gn']), jnp.asarray(c['Cn']),
                                  jnp.asarray(c['Mn']), mask3, gtp_nlp)

    ones_rpm = jnp.ones((B, 50), jnp.float32)
    ones_mpm = jnp.ones((B, 148), jnp.float32)

    return (temporal,
            jnp.asarray(c['m_t']), jnp.asarray(c['rv_t']),
            img_sr, jnp.asarray(c['m_i']), jnp.asarray(c['rv_i']),
            ones_rpm, ones_mpm,
            nlp_sr, jnp.asarray(c['m_n']), jnp.asarray(c['rv_n']),
            rpm3.reshape(B, 129), mpm3.reshape(B, 385))


# speedup vs baseline: 7.6661x; 1.0153x over previous
import functools

import jax
import jax.numpy as jnp
import numpy as np
from jax.experimental import pallas as pl
from jax.experimental.pallas import tpu as pltpu

D_MODEL = 768
B = 4
L_T = 2048


def _pe_table_np(d_model, max_len):
    position = np.arange(max_len)[:, None].astype(np.float32)
    div_term = np.exp(np.arange(0, d_model, 2).astype(np.float32) * (-np.log(10000.0) / d_model))
    pe = np.zeros((max_len, d_model), np.float32)
    pe[:, 0::2] = np.sin(position * div_term)
    pe[:, 1::2] = np.cos(position * div_term)
    return pe


def _1d_sincos_np(embed_dim, pos):
    omega = np.arange(embed_dim // 2, dtype=np.float32) / (embed_dim / 2.0)
    omega = 1.0 / 10000 ** omega
    pos = pos.reshape(-1)
    out = np.einsum('m,d->md', pos, omega)
    return np.concatenate([np.sin(out), np.cos(out)], axis=1)


def _2d_sincos_np(embed_dim, grid_size, cls_token=True):
    grid_h = np.arange(grid_size, dtype=np.float32)
    grid_w = np.arange(grid_size, dtype=np.float32)
    grid = np.meshgrid(grid_w, grid_h)
    grid = np.stack(grid, axis=0).reshape([2, 1, grid_size, grid_size])
    emb_h = _1d_sincos_np(embed_dim // 2, grid[0])
    emb_w = _1d_sincos_np(embed_dim // 2, grid[1])
    pos = np.concatenate([emb_h, emb_w], axis=1).astype(np.float32)
    if cls_token:
        pos = np.concatenate([np.zeros([1, embed_dim], np.float32), pos], axis=0)
    return pos


@functools.lru_cache(maxsize=None)
def _consts():
    mkey = jax.random.key(42)

    def idx(key, shape, num_remain):
        noise = jax.random.uniform(key, shape)
        shuffle = jnp.argsort(noise, axis=-1)
        remain = shuffle[..., :num_remain]
        masked = shuffle[..., num_remain:]
        revert = jnp.argsort(shuffle, axis=-1)
        return (np.asarray(jax.device_get(remain)),
                np.asarray(jax.device_get(masked)),
                np.asarray(jax.device_get(revert)))

    r_t, m_t, rv_t = idx(jax.random.fold_in(mkey, 0), (B, L_T, 4), 2)
    r_i, m_i, rv_i = idx(jax.random.fold_in(mkey, 1), (B, 196), 49)
    r_n, m_n, rv_n = idx(jax.random.fold_in(mkey, 2), (B, 512), 128)

    pe = _pe_table_np(D_MODEL, 4096)
    pos2d = _2d_sincos_np(D_MODEL, 14, True)

    S = np.zeros((B, L_T, 8), np.float32)
    for j in range(2):
        for mm in range(4):
            S[:, :, j * 4 + mm] = (r_t[:, :, j] == mm).astype(np.float32)

    Gi = np.zeros((B, 49, 196), np.float32)
    Gi[np.arange(B)[:, None], np.arange(49)[None, :], r_i] = 1.0
    Ci = pos2d[1:][r_i]

    Gn = np.zeros((B, 128, 512), np.float32)
    Gn[np.arange(B)[:, None], np.arange(128)[None, :], r_n] = 1.0
    Cn = pe[1:513][r_n]
    Mn = np.zeros((B, 384, 512), np.float32)
    Mn[np.arange(B)[:, None], np.arange(384)[None, :], m_n] = 1.0

    return dict(
        r_t=r_t, m_t=m_t, rv_t=rv_t,
        m_i=m_i, rv_i=rv_i, m_n=m_n, rv_n=rv_n,
        pe=pe, pos2d=pos2d,
        S=S, Gi=Gi, Ci=Ci, Gn=Gn, Cn=Cn, Mn=Mn,
    )


_C = _consts()


_LBLK = 512


def _temporal_body(t0, t1, t2, t3, t4, s, pe, w, o):
    peb = pe[...]
    wv = w[...]
    sb = s[0]
    out0 = t0[0] + peb + wv[0]
    x1 = (sb[:, 0:1] * t1[0] + sb[:, 1:2] * t2[0]
          + sb[:, 2:3] * t3[0] + sb[:, 3:4] * t4[0])
    w1 = jax.lax.dot_general(sb[:, 0:4], wv[1:5], (((1,), (0,)), ((), ())),
                             preferred_element_type=jnp.float32)
    x2 = (sb[:, 4:5] * t1[0] + sb[:, 5:6] * t2[0]
          + sb[:, 6:7] * t3[0] + sb[:, 7:8] * t4[0])
    w2 = jax.lax.dot_general(sb[:, 4:8], wv[1:5], (((1,), (0,)), ((), ())),
                             preferred_element_type=jnp.float32)
    o[0] = jnp.stack([out0, x1 + w1 + peb, x2 + w2 + peb], axis=1)


def _temporal(t0, t1, t2, t3, t4, S, pe_t, w_all):
    tb = pl.BlockSpec((1, _LBLK, D_MODEL), lambda l, b: (b, l, 0))
    return pl.pallas_call(
        _temporal_body,
        grid=(L_T // _LBLK, B),
        in_specs=[tb, tb, tb, tb, tb,
                  pl.BlockSpec((1, _LBLK, 8), lambda l, b: (b, l, 0)),
                  pl.BlockSpec((_LBLK, D_MODEL), lambda l, b: (l, 0)),
                  pl.BlockSpec((5, D_MODEL), lambda l, b: (0, 0))],
        out_specs=pl.BlockSpec((1, _LBLK, 3, D_MODEL), lambda l, b: (b, l, 0, 0)),
        out_shape=jax.ShapeDtypeStruct((B, L_T, 3, D_MODEL), jnp.float32),
    )(t0, t1, t2, t3, t4, S, pe_t, w_all)


def _img_body(img, gi, ci, gtp, o):
    rows = jax.lax.dot_general(gi[0], img[0], (((1,), (0,)), ((), ())),
                               preferred_element_type=jnp.float32) + ci[0]
    o[0] = jnp.concatenate([gtp[...], rows], axis=0)


def _img_sr(img, Gi, Ci, gtp):
    return pl.pallas_call(
        _img_body,
        grid=(B,),
        in_specs=[pl.BlockSpec((1, 196, D_MODEL), lambda b: (b, 0, 0)),
                  pl.BlockSpec((1, 49, 196), lambda b: (b, 0, 0)),
                  pl.BlockSpec((1, 49, D_MODEL), lambda b: (b, 0, 0)),
                  pl.BlockSpec((1, D_MODEL), lambda b: (0, 0))],
        out_specs=pl.BlockSpec((1, 50, D_MODEL), lambda b: (b, 0, 0)),
        out_shape=jax.ShapeDtypeStruct((B, 50, D_MODEL), jnp.float32),
    )(img, Gi, Ci, gtp)


def _nlp_body(nlp, gn, cn, mn, msk, gtp, o_sr, o_rpm, o_mpm):
    rows = jax.lax.dot_general(gn[0], nlp[0], (((1,), (0,)), ((), ())),
                               preferred_element_type=jnp.float32) + cn[0]
    o_sr[0] = jnp.concatenate([gtp[...], rows], axis=0)
    mrow = msk[0]
    mv = mrow[:, 1:]
    g0 = mrow[:, 0:1]
    rpm = jax.lax.dot_general(mv, gn[0], (((1,), (1,)), ((), ())),
                              preferred_element_type=jnp.float32)
    mpm = jax.lax.dot_general(mv, mn[0], (((1,), (1,)), ((), ())),
                              preferred_element_type=jnp.float32)
    o_rpm[0] = jnp.concatenate([g0, rpm], axis=1)
    o_mpm[0] = jnp.concatenate([g0, mpm], axis=1)


def _nlp_all(nlp, Gn, Cn, Mn, mask3, gtp):
    return pl.pallas_call(
        _nlp_body,
        grid=(B,),
        in_specs=[pl.BlockSpec((1, 512, D_MODEL), lambda b: (b, 0, 0)),
                  pl.BlockSpec((1, 128, 512), lambda b: (b, 0, 0)),
                  pl.BlockSpec((1, 128, D_MODEL), lambda b: (b, 0, 0)),
                  pl.BlockSpec((1, 384, 512), lambda b: (b, 0, 0)),
                  pl.BlockSpec((1, 1, 513), lambda b: (b, 0, 0)),
                  pl.BlockSpec((1, D_MODEL), lambda b: (0, 0))],
        out_specs=[pl.BlockSpec((1, 129, D_MODEL), lambda b: (b, 0, 0)),
                   pl.BlockSpec((1, 1, 129), lambda b: (b, 0, 0)),
                   pl.BlockSpec((1, 1, 385), lambda b: (b, 0, 0))],
        out_shape=[jax.ShapeDtypeStruct((B, 129, D_MODEL), jnp.float32),
                   jax.ShapeDtypeStruct((B, 1, 129), jnp.float32),
                   jax.ShapeDtypeStruct((B, 1, 385), jnp.float32)],
    )(nlp, Gn, Cn, Mn, mask3, gtp)


def kernel(t0, t1, t2, t3, t4, img, nlp, nlp_revert_padding_mask,
           W_temporal_mod, W_total_mod, global_token):
    c = _C
    pe_t = jnp.asarray(c['pe'][:L_T])
    w_all = W_temporal_mod + W_total_mod[0][None, :]

    temporal = _temporal(t0, t1, t2, t3, t4, jnp.asarray(c['S']), pe_t, w_all)

    gt = global_token[0]
    gtp_img = gt + jnp.asarray(c['pos2d'][0:1])
    img_sr = _img_sr(img, jnp.asarray(c['Gi']), jnp.asarray(c['Ci']), gtp_img)

    gtp_nlp = gt + jnp.asarray(c['pe'][0:1])
    mask3 = nlp_revert_padding_mask[:, None, :]
    nlp_sr, rpm3, mpm3 = _nlp_all(nlp, jnp.asarray(c['Gn']), jnp.asarray(c['Cn']),
                                  jnp.asarray(c['Mn']), mask3, gtp_nlp)

    ones_rpm = jnp.ones((B, 50), jnp.float32)
    ones_mpm = jnp.ones((B, 148), jnp.float32)

    return (temporal,
            jnp.asarray(c['m_t']), jnp.asarray(c['rv_t']),
            img_sr, jnp.asarray(c['m_i']), jnp.asarray(c['rv_i']),
            ones_rpm, ones_mpm,
            nlp_sr, jnp.asarray(c['m_n']), jnp.asarray(c['rv_n']),
            rpm3.reshape(B, 129), mpm3.reshape(B, 385))
